# baseline (speedup 1.0000x reference)
import functools

import jax
import jax.numpy as jnp
from jax import lax
from jax.experimental import pallas as pl
from jax.experimental.pallas import tpu as pltpu
from jax.experimental.pallas import tpu_sc as plsc

N = 10000
E = 320000
D = 128
K = 10
ALPHA = 0.1

CH = 128
TILES = 32
CHUNKS_PER_TILE = 80
EP_TILE = CHUNKS_PER_TILE * CH
E_PAD = EP_TILE * TILES
NB = 2
NP = 10240
SAC = 10008
ROWS_T = NP // 16

_mesh = plsc.VectorSubcoreMesh(core_axis_name="c", subcore_axis_name="s")


@functools.partial(
    pl.kernel,
    mesh=_mesh,
    out_type=[
        jax.ShapeDtypeStruct((NP, 16), jnp.float32),
        jax.ShapeDtypeStruct((NP, 16), jnp.float32),
    ],
    scratch_types=[
        pltpu.VMEM_SHARED((NP, 16), jnp.float32),
        pltpu.VMEM((CH,), jnp.int32),
        pltpu.VMEM((CH, 16), jnp.float32),
    ],
)
def _deg_kernel(dst_hbm, ones_hbm, degA, degB, sh_deg, idx_v, ones_v):
    c = lax.axis_index("c")
    s = lax.axis_index("s")
    wid = c * 16 + s
    pltpu.sync_copy(ones_hbm.at[pl.ds(s * ROWS_T, ROWS_T)],
                    sh_deg.at[pl.ds(s * ROWS_T, ROWS_T)])
    pltpu.sync_copy(ones_hbm.at[pl.ds(0, CH)], ones_v)
    plsc.subcore_barrier()

    def body(j, carry):
        pltpu.sync_copy(dst_hbm.at[wid].at[j], idx_v)
        pltpu.sync_copy(ones_v, sh_deg.at[idx_v], add=True)
        return carry

    lax.fori_loop(0, CHUNKS_PER_TILE, body, 0)
    plsc.subcore_barrier()

    @pl.when(c == 0)
    def _():
        pltpu.sync_copy(sh_deg.at[pl.ds(s * ROWS_T, ROWS_T)],
                        degA.at[pl.ds(s * ROWS_T, ROWS_T)])

    @pl.when(c == 1)
    def _():
        pltpu.sync_copy(sh_deg.at[pl.ds(s * ROWS_T, ROWS_T)],
                        degB.at[pl.ds(s * ROWS_T, ROWS_T)])


@functools.partial(
    pl.kernel,
    mesh=_mesh,
    out_type=[
        jax.ShapeDtypeStruct((NP, D), jnp.float32),
        jax.ShapeDtypeStruct((NP, D), jnp.float32),
    ],
    scratch_types=(
        [pltpu.VMEM_SHARED((NP, D), jnp.float32)]
        + [pltpu.VMEM((CHUNKS_PER_TILE, CH), jnp.int32)]
        + [pltpu.VMEM((CH,), jnp.int32)] * NB
        + [pltpu.VMEM((CH, D), jnp.float32)] * NB
        + [pltpu.SemaphoreType.DMA] * (2 * NB)
    ),
)
def _scatter_step(g_hbm, src_hbm, dst_hbm, aggA, aggB, sh_agg, idx_sb, *rest):
    isc = rest[0:NB]
    rows = rest[NB:2 * NB]
    sem_g = rest[2 * NB:3 * NB]
    sem_ic = rest[3 * NB:4 * NB]
    c = lax.axis_index("c")
    s = lax.axis_index("s")
    wid = c * 16 + s
    pltpu.sync_copy(g_hbm.at[pl.ds(s * ROWS_T, ROWS_T)],
                    sh_agg.at[pl.ds(s * ROWS_T, ROWS_T)])
    pltpu.sync_copy(src_hbm.at[wid], idx_sb)
    plsc.subcore_barrier()

    def body(j, carry):
        pltpu.sync_copy(dst_hbm.at[wid].at[j], isc[0])
        pltpu.async_copy(g_hbm.at[idx_sb.at[j]], rows[0], sem_g[0]).wait()
        pltpu.sync_copy(rows[0], sh_agg.at[isc[0]], add=True)
        return carry

    lax.fori_loop(0, CHUNKS_PER_TILE, body, 0)
    plsc.subcore_barrier()

    @pl.when(c == 0)
    def _():
        pltpu.sync_copy(sh_agg.at[pl.ds(s * ROWS_T, ROWS_T)],
                        aggA.at[pl.ds(s * ROWS_T, ROWS_T)])

    @pl.when(c == 1)
    def _():
        pltpu.sync_copy(sh_agg.at[pl.ds(s * ROWS_T, ROWS_T)],
                        aggB.at[pl.ds(s * ROWS_T, ROWS_T)])


def _mlp_body(x_ref, w1_ref, b1_ref, w2_ref, b2_ref, da_ref, db_ref,
              x0_ref, g0_ref, dinv_ref):
    xb = x_ref[...]
    h = lax.dot_general(xb, w1_ref[...], (((1,), (1,)), ((), ())),
                        preferred_element_type=jnp.float32)
    h = jnp.maximum(h + b1_ref[...], 0.0)
    h = lax.dot_general(h, w2_ref[...], (((1,), (1,)), ((), ())),
                        preferred_element_type=jnp.float32) + b2_ref[...]
    deg = da_ref[...][:, :1] + db_ref[...][:, :1] - 1.0
    dinv = lax.rsqrt(deg)
    x0_ref[...] = h
    g0_ref[...] = h * dinv
    dinv_ref[...] = dinv


def _combine_body(aggA_ref, aggB_ref, g_ref, x0_ref, dinv_ref, h_ref, gn_ref):
    agg = aggA_ref[...] + aggB_ref[...] - g_ref[...]
    dinv = dinv_ref[...]
    h = (1.0 - ALPHA) * (dinv * agg) + ALPHA * x0_ref[...]
    h_ref[...] = h
    gn_ref[...] = dinv * h


_BLK = 1024
_GRID = NP // _BLK

_mlp_call = pl.pallas_call(
    _mlp_body,
    grid=(_GRID,),
    in_specs=[
        pl.BlockSpec((_BLK, D), lambda i: (i, 0)),
        pl.BlockSpec((D, D), lambda i: (0, 0)),
        pl.BlockSpec((1, D), lambda i: (0, 0)),
        pl.BlockSpec((D, D), lambda i: (0, 0)),
        pl.BlockSpec((1, D), lambda i: (0, 0)),
        pl.BlockSpec((_BLK, 16), lambda i: (i, 0)),
        pl.BlockSpec((_BLK, 16), lambda i: (i, 0)),
    ],
    out_specs=[
        pl.BlockSpec((_BLK, D), lambda i: (i, 0)),
        pl.BlockSpec((_BLK, D), lambda i: (i, 0)),
        pl.BlockSpec((_BLK, 1), lambda i: (i, 0)),
    ],
    out_shape=[
        jax.ShapeDtypeStruct((NP, D), jnp.float32),
        jax.ShapeDtypeStruct((NP, D), jnp.float32),
        jax.ShapeDtypeStruct((NP, 1), jnp.float32),
    ],
)

_combine_call = pl.pallas_call(
    _combine_body,
    grid=(_GRID,),
    in_specs=[
        pl.BlockSpec((_BLK, D), lambda i: (i, 0)),
        pl.BlockSpec((_BLK, D), lambda i: (i, 0)),
        pl.BlockSpec((_BLK, D), lambda i: (i, 0)),
        pl.BlockSpec((_BLK, D), lambda i: (i, 0)),
        pl.BlockSpec((_BLK, 1), lambda i: (i, 0)),
    ],
    out_specs=[
        pl.BlockSpec((_BLK, D), lambda i: (i, 0)),
        pl.BlockSpec((_BLK, D), lambda i: (i, 0)),
    ],
    out_shape=[
        jax.ShapeDtypeStruct((NP, D), jnp.float32),
        jax.ShapeDtypeStruct((NP, D), jnp.float32),
    ],
)


def kernel(x, edge_index, W1, b1, W2, b2):
    src = edge_index[0]
    dst = edge_index[1]
    pad = E_PAD - E
    src_p = jnp.concatenate([src, jnp.zeros((pad,), jnp.int32)])
    dst_p = jnp.concatenate([dst, jnp.full((pad,), SAC, jnp.int32)])
    src_p = src_p.reshape(TILES, CHUNKS_PER_TILE, CH)
    dst_p = dst_p.reshape(TILES, CHUNKS_PER_TILE, CH)
    ones16 = jnp.ones((NP, 16), jnp.float32)
    x_p = jnp.pad(x, ((0, NP - N), (0, 0)))

    degA, degB = _deg_kernel(dst_p, ones16)
    x0, g, dinv = _mlp_call(x_p, W1, b1.reshape(1, D), W2, b2.reshape(1, D),
                            degA, degB)
    h = x0
    for _ in range(K):
        aggA, aggB = _scatter_step(g, src_p, dst_p)
        h, g = _combine_call(aggA, aggB, g, x0, dinv)
    return h[:N]


# speedup vs baseline: 1.3008x; 1.3008x over previous
import functools

import jax
import jax.numpy as jnp
from jax import lax
from jax.experimental import pallas as pl
from jax.experimental.pallas import tpu as pltpu
from jax.experimental.pallas import tpu_sc as plsc

N = 10000
E = 320000
D = 128
K = 10
ALPHA = 0.1

CH = 128
TILES = 32
CHUNKS_PER_TILE = 79
EP_TILE = CHUNKS_PER_TILE * CH
E_PAD = EP_TILE * TILES
NP = 10240
SAC = 10008
ROWS_T = NP // 16

_mesh = plsc.VectorSubcoreMesh(core_axis_name="c", subcore_axis_name="s")


@functools.partial(
    pl.kernel,
    mesh=_mesh,
    out_type=[
        jax.ShapeDtypeStruct((NP, 16), jnp.float32),
        jax.ShapeDtypeStruct((NP, 16), jnp.float32),
    ],
    scratch_types=[
        pltpu.VMEM_SHARED((NP, 16), jnp.float32),
        pltpu.VMEM((CH,), jnp.int32),
        pltpu.VMEM((CH, 16), jnp.float32),
    ],
)
def _deg_kernel(dst_hbm, ones_hbm, degA, degB, sh_deg, idx_v, ones_v):
    c = lax.axis_index("c")
    s = lax.axis_index("s")
    wid = c * 16 + s
    pltpu.sync_copy(ones_hbm.at[pl.ds(s * ROWS_T, ROWS_T)],
                    sh_deg.at[pl.ds(s * ROWS_T, ROWS_T)])
    pltpu.sync_copy(ones_hbm.at[pl.ds(0, CH)], ones_v)
    plsc.subcore_barrier()

    base0 = wid * EP_TILE

    def body(j, carry):
        pltpu.sync_copy(dst_hbm.at[pl.ds(base0 + j * CH, CH)], idx_v)
        pltpu.sync_copy(ones_v, sh_deg.at[idx_v], add=True)
        return carry

    lax.fori_loop(0, CHUNKS_PER_TILE, body, 0)
    plsc.subcore_barrier()

    @pl.when(c == 0)
    def _():
        pltpu.sync_copy(sh_deg.at[pl.ds(s * ROWS_T, ROWS_T)],
                        degA.at[pl.ds(s * ROWS_T, ROWS_T)])

    @pl.when(c == 1)
    def _():
        pltpu.sync_copy(sh_deg.at[pl.ds(s * ROWS_T, ROWS_T)],
                        degB.at[pl.ds(s * ROWS_T, ROWS_T)])


@functools.partial(
    pl.kernel,
    mesh=_mesh,
    out_type=[
        jax.ShapeDtypeStruct((NP, D), jnp.float32),
        jax.ShapeDtypeStruct((NP, D), jnp.float32),
    ],
    scratch_types=[
        pltpu.VMEM_SHARED((NP, D), jnp.float32),
        pltpu.VMEM((CH,), jnp.int32),
        pltpu.VMEM((CH,), jnp.int32),
        pltpu.VMEM((CH, D), jnp.float32),
        pltpu.SemaphoreType.DMA,
    ],
)
def _scatter_step(g_hbm, src_hbm, dst_hbm, aggA, aggB,
                  sh_agg, idx_s, idx_d, rows, sem):
    c = lax.axis_index("c")
    s = lax.axis_index("s")
    wid = c * 16 + s
    pltpu.sync_copy(g_hbm.at[pl.ds(s * ROWS_T, ROWS_T)],
                    sh_agg.at[pl.ds(s * ROWS_T, ROWS_T)])
    plsc.subcore_barrier()
    base0 = wid * EP_TILE

    def body(j, carry):
        b = base0 + j * CH
        pltpu.sync_copy(src_hbm.at[pl.ds(b, CH)], idx_s)
        pltpu.sync_copy(dst_hbm.at[pl.ds(b, CH)], idx_d)
        pltpu.async_copy(g_hbm.at[idx_s], rows, sem).wait()
        pltpu.sync_copy(rows, sh_agg.at[idx_d], add=True)
        return carry

    lax.fori_loop(0, CHUNKS_PER_TILE, body, 0)
    plsc.subcore_barrier()

    @pl.when(c == 0)
    def _():
        pltpu.sync_copy(sh_agg.at[pl.ds(s * ROWS_T, ROWS_T)],
                        aggA.at[pl.ds(s * ROWS_T, ROWS_T)])

    @pl.when(c == 1)
    def _():
        pltpu.sync_copy(sh_agg.at[pl.ds(s * ROWS_T, ROWS_T)],
                        aggB.at[pl.ds(s * ROWS_T, ROWS_T)])


def _mlp_body(x_ref, w1_ref, b1_ref, w2_ref, b2_ref, da_ref, db_ref,
              x0_ref, g0_ref, dinv_ref):
    xb = x_ref[...]
    h = lax.dot_general(xb, w1_ref[...], (((1,), (1,)), ((), ())),
                        preferred_element_type=jnp.float32)
    h = jnp.maximum(h + b1_ref[...], 0.0)
    h = lax.dot_general(h, w2_ref[...], (((1,), (1,)), ((), ())),
                        preferred_element_type=jnp.float32) + b2_ref[...]
    deg = da_ref[...][:, :1] + db_ref[...][:, :1] - 1.0
    dinv = lax.rsqrt(deg)
    x0_ref[...] = h
    g0_ref[...] = h * dinv
    dinv_ref[...] = dinv


def _combine_body(aggA_ref, aggB_ref, g_ref, x0_ref, dinv_ref, h_ref, gn_ref):
    agg = aggA_ref[...] + aggB_ref[...] - g_ref[...]
    dinv = dinv_ref[...]
    h = (1.0 - ALPHA) * (dinv * agg) + ALPHA * x0_ref[...]
    h_ref[...] = h
    gn_ref[...] = dinv * h


_BLK = 1024
_GRID = NP // _BLK

_mlp_call = pl.pallas_call(
    _mlp_body,
    grid=(_GRID,),
    in_specs=[
        pl.BlockSpec((_BLK, D), lambda i: (i, 0)),
        pl.BlockSpec((D, D), lambda i: (0, 0)),
        pl.BlockSpec((1, D), lambda i: (0, 0)),
        pl.BlockSpec((D, D), lambda i: (0, 0)),
        pl.BlockSpec((1, D), lambda i: (0, 0)),
        pl.BlockSpec((_BLK, 16), lambda i: (i, 0)),
        pl.BlockSpec((_BLK, 16), lambda i: (i, 0)),
    ],
    out_specs=[
        pl.BlockSpec((_BLK, D), lambda i: (i, 0)),
        pl.BlockSpec((_BLK, D), lambda i: (i, 0)),
        pl.BlockSpec((_BLK, 1), lambda i: (i, 0)),
    ],
    out_shape=[
        jax.ShapeDtypeStruct((NP, D), jnp.float32),
        jax.ShapeDtypeStruct((NP, D), jnp.float32),
        jax.ShapeDtypeStruct((NP, 1), jnp.float32),
    ],
)

_combine_call = pl.pallas_call(
    _combine_body,
    grid=(_GRID,),
    in_specs=[
        pl.BlockSpec((_BLK, D), lambda i: (i, 0)),
        pl.BlockSpec((_BLK, D), lambda i: (i, 0)),
        pl.BlockSpec((_BLK, D), lambda i: (i, 0)),
        pl.BlockSpec((_BLK, D), lambda i: (i, 0)),
        pl.BlockSpec((_BLK, 1), lambda i: (i, 0)),
    ],
    out_specs=[
        pl.BlockSpec((_BLK, D), lambda i: (i, 0)),
        pl.BlockSpec((_BLK, D), lambda i: (i, 0)),
    ],
    out_shape=[
        jax.ShapeDtypeStruct((NP, D), jnp.float32),
        jax.ShapeDtypeStruct((NP, D), jnp.float32),
    ],
)


def kernel(x, edge_index, W1, b1, W2, b2):
    src = edge_index[0]
    dst = edge_index[1]
    pad = E_PAD - E
    src_p = jnp.concatenate([src, jnp.zeros((pad,), jnp.int32)])
    dst_p = jnp.concatenate([dst, jnp.full((pad,), SAC, jnp.int32)])
    ones16 = jnp.ones((NP, 16), jnp.float32)
    x_p = jnp.pad(x, ((0, NP - N), (0, 0)))

    degA, degB = _deg_kernel(dst_p, ones16)
    x0, g, dinv = _mlp_call(x_p, W1, b1.reshape(1, D), W2, b2.reshape(1, D),
                            degA, degB)
    h = x0
    for _ in range(K):
        aggA, aggB = _scatter_step(g, src_p, dst_p)
        h, g = _combine_call(aggA, aggB, g, x0, dinv)
    return h[:N]


# speedup vs baseline: 1.4013x; 1.0772x over previous
import functools

import jax
import jax.numpy as jnp
from jax import lax
from jax.experimental import pallas as pl
from jax.experimental.pallas import tpu as pltpu
from jax.experimental.pallas import tpu_sc as plsc

N = 10000
E = 320000
D = 128
K = 10
ALPHA = 0.1

CH = 128
TILES = 32
CHUNKS_PER_TILE = 79
EP_TILE = CHUNKS_PER_TILE * CH
E_PAD = EP_TILE * TILES
NP = 10240
SAC = 10008
ROWS_T = NP // 16

_mesh = plsc.VectorSubcoreMesh(core_axis_name="c", subcore_axis_name="s")


@functools.partial(
    pl.kernel,
    mesh=_mesh,
    out_type=[
        jax.ShapeDtypeStruct((NP, 16), jnp.float32),
        jax.ShapeDtypeStruct((NP, 16), jnp.float32),
    ],
    scratch_types=[
        pltpu.VMEM_SHARED((NP, 16), jnp.float32),
        pltpu.VMEM((CH,), jnp.int32),
        pltpu.VMEM((CH, 16), jnp.float32),
    ],
)
def _deg_kernel(dst_hbm, ones_hbm, degA, degB, sh_deg, idx_v, ones_v):
    c = lax.axis_index("c")
    s = lax.axis_index("s")
    wid = c * 16 + s
    pltpu.sync_copy(ones_hbm.at[pl.ds(s * ROWS_T, ROWS_T)],
                    sh_deg.at[pl.ds(s * ROWS_T, ROWS_T)])
    pltpu.sync_copy(ones_hbm.at[pl.ds(0, CH)], ones_v)
    plsc.subcore_barrier()

    base0 = wid * EP_TILE

    def body(j, carry):
        pltpu.sync_copy(dst_hbm.at[pl.ds(base0 + j * CH, CH)], idx_v)
        pltpu.sync_copy(ones_v, sh_deg.at[idx_v], add=True)
        return carry

    lax.fori_loop(0, CHUNKS_PER_TILE, body, 0)
    plsc.subcore_barrier()

    @pl.when(c == 0)
    def _():
        pltpu.sync_copy(sh_deg.at[pl.ds(s * ROWS_T, ROWS_T)],
                        degA.at[pl.ds(s * ROWS_T, ROWS_T)])

    @pl.when(c == 1)
    def _():
        pltpu.sync_copy(sh_deg.at[pl.ds(s * ROWS_T, ROWS_T)],
                        degB.at[pl.ds(s * ROWS_T, ROWS_T)])


@functools.partial(
    pl.kernel,
    mesh=_mesh,
    out_type=[
        jax.ShapeDtypeStruct((NP, D), jnp.float32),
        jax.ShapeDtypeStruct((NP, D), jnp.float32),
    ],
    scratch_types=[
        pltpu.VMEM_SHARED((NP, D), jnp.float32),
        pltpu.VMEM((CH,), jnp.int32),
        pltpu.VMEM((CH,), jnp.int32),
        pltpu.VMEM((CH, D), jnp.float32),
        pltpu.SemaphoreType.DMA,
        pltpu.SemaphoreType.DMA,
    ],
)
def _scatter_step(g_hbm, src_hbm, dst_hbm, aggA, aggB,
                  sh_agg, idx_s, idx_d, rows, sem, sem2):
    c = lax.axis_index("c")
    s = lax.axis_index("s")
    wid = c * 16 + s
    pltpu.sync_copy(g_hbm.at[pl.ds(s * ROWS_T, ROWS_T)],
                    sh_agg.at[pl.ds(s * ROWS_T, ROWS_T)])
    plsc.subcore_barrier()
    base0 = wid * EP_TILE

    def body(j, carry):
        b = base0 + j * CH
        pltpu.sync_copy(src_hbm.at[pl.ds(b, CH)], idx_s)
        gather = pltpu.async_copy(g_hbm.at[idx_s], rows, sem)
        pltpu.async_copy(dst_hbm.at[pl.ds(b, CH)], idx_d, sem2)
        gather.wait()
        pltpu.make_async_copy(dst_hbm.at[pl.ds(b, CH)], idx_d, sem2).wait()
        pltpu.sync_copy(rows, sh_agg.at[idx_d], add=True)
        return carry

    lax.fori_loop(0, CHUNKS_PER_TILE, body, 0)
    plsc.subcore_barrier()

    @pl.when(c == 0)
    def _():
        pltpu.sync_copy(sh_agg.at[pl.ds(s * ROWS_T, ROWS_T)],
                        aggA.at[pl.ds(s * ROWS_T, ROWS_T)])

    @pl.when(c == 1)
    def _():
        pltpu.sync_copy(sh_agg.at[pl.ds(s * ROWS_T, ROWS_T)],
                        aggB.at[pl.ds(s * ROWS_T, ROWS_T)])


def _mlp_body(x_ref, w1_ref, b1_ref, w2_ref, b2_ref, da_ref, db_ref,
              x0_ref, g0_ref, dinv_ref):
    xb = x_ref[...]
    h = lax.dot_general(xb, w1_ref[...], (((1,), (1,)), ((), ())),
                        preferred_element_type=jnp.float32)
    h = jnp.maximum(h + b1_ref[...], 0.0)
    h = lax.dot_general(h, w2_ref[...], (((1,), (1,)), ((), ())),
                        preferred_element_type=jnp.float32) + b2_ref[...]
    deg = da_ref[...][:, :1] + db_ref[...][:, :1] - 1.0
    dinv = lax.rsqrt(deg)
    x0_ref[...] = h
    g0_ref[...] = h * dinv
    dinv_ref[...] = dinv


def _combine_body(aggA_ref, aggB_ref, g_ref, x0_ref, dinv_ref, h_ref, gn_ref):
    agg = aggA_ref[...] + aggB_ref[...] - g_ref[...]
    dinv = dinv_ref[...]
    h = (1.0 - ALPHA) * (dinv * agg) + ALPHA * x0_ref[...]
    h_ref[...] = h
    gn_ref[...] = dinv * h


_BLK = 1024
_GRID = NP // _BLK

_mlp_call = pl.pallas_call(
    _mlp_body,
    grid=(_GRID,),
    in_specs=[
        pl.BlockSpec((_BLK, D), lambda i: (i, 0)),
        pl.BlockSpec((D, D), lambda i: (0, 0)),
        pl.BlockSpec((1, D), lambda i: (0, 0)),
        pl.BlockSpec((D, D), lambda i: (0, 0)),
        pl.BlockSpec((1, D), lambda i: (0, 0)),
        pl.BlockSpec((_BLK, 16), lambda i: (i, 0)),
        pl.BlockSpec((_BLK, 16), lambda i: (i, 0)),
    ],
    out_specs=[
        pl.BlockSpec((_BLK, D), lambda i: (i, 0)),
        pl.BlockSpec((_BLK, D), lambda i: (i, 0)),
        pl.BlockSpec((_BLK, 1), lambda i: (i, 0)),
    ],
    out_shape=[
        jax.ShapeDtypeStruct((NP, D), jnp.float32),
        jax.ShapeDtypeStruct((NP, D), jnp.float32),
        jax.ShapeDtypeStruct((NP, 1), jnp.float32),
    ],
)

_combine_call = pl.pallas_call(
    _combine_body,
    grid=(_GRID,),
    in_specs=[
        pl.BlockSpec((_BLK, D), lambda i: (i, 0)),
        pl.BlockSpec((_BLK, D), lambda i: (i, 0)),
        pl.BlockSpec((_BLK, D), lambda i: (i, 0)),
        pl.BlockSpec((_BLK, D), lambda i: (i, 0)),
        pl.BlockSpec((_BLK, 1), lambda i: (i, 0)),
    ],
    out_specs=[
        pl.BlockSpec((_BLK, D), lambda i: (i, 0)),
        pl.BlockSpec((_BLK, D), lambda i: (i, 0)),
    ],
    out_shape=[
        jax.ShapeDtypeStruct((NP, D), jnp.float32),
        jax.ShapeDtypeStruct((NP, D), jnp.float32),
    ],
)


def kernel(x, edge_index, W1, b1, W2, b2):
    src = edge_index[0]
    dst = edge_index[1]
    pad = E_PAD - E
    src_p = jnp.concatenate([src, jnp.zeros((pad,), jnp.int32)])
    dst_p = jnp.concatenate([dst, jnp.full((pad,), SAC, jnp.int32)])
    ones16 = jnp.ones((NP, 16), jnp.float32)
    x_p = jnp.pad(x, ((0, NP - N), (0, 0)))

    degA, degB = _deg_kernel(dst_p, ones16)
    x0, g, dinv = _mlp_call(x_p, W1, b1.reshape(1, D), W2, b2.reshape(1, D),
                            degA, degB)
    h = x0
    for _ in range(K):
        aggA, aggB = _scatter_step(g, src_p, dst_p)
        h, g = _combine_call(aggA, aggB, g, x0, dinv)
    return h[:N]


# speedup vs baseline: 1.5217x; 1.0859x over previous
import functools

import jax
import jax.numpy as jnp
from jax import lax
from jax.experimental import pallas as pl
from jax.experimental.pallas import tpu as pltpu
from jax.experimental.pallas import tpu_sc as plsc

N = 10000
E = 320000
D = 128
K = 10
ALPHA = 0.1

CH = 128
TILES = 32
CHUNKS_PER_TILE = 79
EP_TILE = CHUNKS_PER_TILE * CH
E_PAD = EP_TILE * TILES
NP = 10240
SAC = 10008
ROWS_T = NP // 16

_mesh = plsc.VectorSubcoreMesh(core_axis_name="c", subcore_axis_name="s")


@functools.partial(
    pl.kernel,
    mesh=_mesh,
    out_type=[
        jax.ShapeDtypeStruct((NP, 16), jnp.float32),
        jax.ShapeDtypeStruct((NP, 16), jnp.float32),
    ],
    scratch_types=[
        pltpu.VMEM_SHARED((NP, 16), jnp.float32),
        pltpu.VMEM((CH,), jnp.int32),
        pltpu.VMEM((CH, 16), jnp.float32),
    ],
)
def _deg_kernel(dst_hbm, ones_hbm, degA, degB, sh_deg, idx_v, ones_v):
    c = lax.axis_index("c")
    s = lax.axis_index("s")
    wid = c * 16 + s
    pltpu.sync_copy(ones_hbm.at[pl.ds(s * ROWS_T, ROWS_T)],
                    sh_deg.at[pl.ds(s * ROWS_T, ROWS_T)])
    pltpu.sync_copy(ones_hbm.at[pl.ds(0, CH)], ones_v)
    plsc.subcore_barrier()

    base0 = wid * EP_TILE

    def body(j, carry):
        pltpu.sync_copy(dst_hbm.at[pl.ds(base0 + j * CH, CH)], idx_v)
        pltpu.sync_copy(ones_v, sh_deg.at[idx_v], add=True)
        return carry

    lax.fori_loop(0, CHUNKS_PER_TILE, body, 0)
    plsc.subcore_barrier()

    @pl.when(c == 0)
    def _():
        pltpu.sync_copy(sh_deg.at[pl.ds(s * ROWS_T, ROWS_T)],
                        degA.at[pl.ds(s * ROWS_T, ROWS_T)])

    @pl.when(c == 1)
    def _():
        pltpu.sync_copy(sh_deg.at[pl.ds(s * ROWS_T, ROWS_T)],
                        degB.at[pl.ds(s * ROWS_T, ROWS_T)])


@functools.partial(
    pl.kernel,
    mesh=_mesh,
    out_type=[
        jax.ShapeDtypeStruct((NP, D), jnp.float32),
        jax.ShapeDtypeStruct((NP, D), jnp.float32),
    ],
    scratch_types=[
        pltpu.VMEM_SHARED((NP, D), jnp.float32),
        pltpu.VMEM((CH,), jnp.int32),
        pltpu.VMEM((CH,), jnp.int32),
        pltpu.VMEM((CH, D), jnp.float32),
        pltpu.SemaphoreType.DMA,
        pltpu.SemaphoreType.DMA,
        pltpu.SemaphoreType.DMA,
    ],
)
def _scatter_step(g_hbm, src_hbm, dst_hbm, aggA, aggB,
                  sh_agg, idx_s, idx_d, rows, sem, sem2, sem3):
    c = lax.axis_index("c")
    s = lax.axis_index("s")
    wid = c * 16 + s
    pltpu.sync_copy(g_hbm.at[pl.ds(s * ROWS_T, ROWS_T)],
                    sh_agg.at[pl.ds(s * ROWS_T, ROWS_T)])
    plsc.subcore_barrier()
    base0 = wid * EP_TILE

    pltpu.sync_copy(src_hbm.at[pl.ds(base0, CH)], idx_s)

    def body(j, carry):
        b = base0 + j * CH
        gather = pltpu.async_copy(g_hbm.at[idx_s], rows, sem)
        pltpu.async_copy(dst_hbm.at[pl.ds(b, CH)], idx_d, sem2)
        gather.wait()
        bn = base0 + jnp.minimum(j + 1, CHUNKS_PER_TILE - 1) * CH
        pltpu.async_copy(src_hbm.at[pl.ds(bn, CH)], idx_s, sem3)
        pltpu.make_async_copy(dst_hbm.at[pl.ds(b, CH)], idx_d, sem2).wait()
        pltpu.sync_copy(rows, sh_agg.at[idx_d], add=True)
        pltpu.make_async_copy(src_hbm.at[pl.ds(bn, CH)], idx_s, sem3).wait()
        return carry

    lax.fori_loop(0, CHUNKS_PER_TILE, body, 0)
    plsc.subcore_barrier()

    @pl.when(c == 0)
    def _():
        pltpu.sync_copy(sh_agg.at[pl.ds(s * ROWS_T, ROWS_T)],
                        aggA.at[pl.ds(s * ROWS_T, ROWS_T)])

    @pl.when(c == 1)
    def _():
        pltpu.sync_copy(sh_agg.at[pl.ds(s * ROWS_T, ROWS_T)],
                        aggB.at[pl.ds(s * ROWS_T, ROWS_T)])


def _mlp_body(x_ref, w1_ref, b1_ref, w2_ref, b2_ref, da_ref, db_ref,
              x0_ref, g0_ref, dinv_ref):
    xb = x_ref[...]
    h = lax.dot_general(xb, w1_ref[...], (((1,), (1,)), ((), ())),
                        preferred_element_type=jnp.float32)
    h = jnp.maximum(h + b1_ref[...], 0.0)
    h = lax.dot_general(h, w2_ref[...], (((1,), (1,)), ((), ())),
                        preferred_element_type=jnp.float32) + b2_ref[...]
    deg = da_ref[...][:, :1] + db_ref[...][:, :1] - 1.0
    dinv = lax.rsqrt(deg)
    x0_ref[...] = h
    g0_ref[...] = h * dinv
    dinv_ref[...] = dinv


def _combine_body(aggA_ref, aggB_ref, g_ref, x0_ref, dinv_ref, h_ref, gn_ref):
    agg = aggA_ref[...] + aggB_ref[...] - g_ref[...]
    dinv = dinv_ref[...]
    h = (1.0 - ALPHA) * (dinv * agg) + ALPHA * x0_ref[...]
    h_ref[...] = h
    gn_ref[...] = dinv * h


_BLK = 1024
_GRID = NP // _BLK

_mlp_call = pl.pallas_call(
    _mlp_body,
    grid=(_GRID,),
    in_specs=[
        pl.BlockSpec((_BLK, D), lambda i: (i, 0)),
        pl.BlockSpec((D, D), lambda i: (0, 0)),
        pl.BlockSpec((1, D), lambda i: (0, 0)),
        pl.BlockSpec((D, D), lambda i: (0, 0)),
        pl.BlockSpec((1, D), lambda i: (0, 0)),
        pl.BlockSpec((_BLK, 16), lambda i: (i, 0)),
        pl.BlockSpec((_BLK, 16), lambda i: (i, 0)),
    ],
    out_specs=[
        pl.BlockSpec((_BLK, D), lambda i: (i, 0)),
        pl.BlockSpec((_BLK, D), lambda i: (i, 0)),
        pl.BlockSpec((_BLK, 1), lambda i: (i, 0)),
    ],
    out_shape=[
        jax.ShapeDtypeStruct((NP, D), jnp.float32),
        jax.ShapeDtypeStruct((NP, D), jnp.float32),
        jax.ShapeDtypeStruct((NP, 1), jnp.float32),
    ],
)

_combine_call = pl.pallas_call(
    _combine_body,
    grid=(_GRID,),
    in_specs=[
        pl.BlockSpec((_BLK, D), lambda i: (i, 0)),
        pl.BlockSpec((_BLK, D), lambda i: (i, 0)),
        pl.BlockSpec((_BLK, D), lambda i: (i, 0)),
        pl.BlockSpec((_BLK, D), lambda i: (i, 0)),
        pl.BlockSpec((_BLK, 1), lambda i: (i, 0)),
    ],
    out_specs=[
        pl.BlockSpec((_BLK, D), lambda i: (i, 0)),
        pl.BlockSpec((_BLK, D), lambda i: (i, 0)),
    ],
    out_shape=[
        jax.ShapeDtypeStruct((NP, D), jnp.float32),
        jax.ShapeDtypeStruct((NP, D), jnp.float32),
    ],
)


def kernel(x, edge_index, W1, b1, W2, b2):
    src = edge_index[0]
    dst = edge_index[1]
    pad = E_PAD - E
    src_p = jnp.concatenate([src, jnp.zeros((pad,), jnp.int32)])
    dst_p = jnp.concatenate([dst, jnp.full((pad,), SAC, jnp.int32)])
    ones16 = jnp.ones((NP, 16), jnp.float32)
    x_p = jnp.pad(x, ((0, NP - N), (0, 0)))

    degA, degB = _deg_kernel(dst_p, ones16)
    x0, g, dinv = _mlp_call(x_p, W1, b1.reshape(1, D), W2, b2.reshape(1, D),
                            degA, degB)
    h = x0
    for _ in range(K):
        aggA, aggB = _scatter_step(g, src_p, dst_p)
        h, g = _combine_call(aggA, aggB, g, x0, dinv)
    return h[:N]


# speedup vs baseline: 1.7181x; 1.1290x over previous
import functools

import jax
import jax.numpy as jnp
from jax import lax
from jax.experimental import pallas as pl
from jax.experimental.pallas import tpu as pltpu
from jax.experimental.pallas import tpu_sc as plsc

N = 10000
E = 320000
D = 128
K = 10
ALPHA = 0.1

CH = 128
TILES = 32
CHUNKS_PER_TILE = 79
EP_TILE = CHUNKS_PER_TILE * CH
E_PAD = EP_TILE * TILES
NP = 10240
SAC = 10008
ROWS_T = NP // 16

_mesh = plsc.VectorSubcoreMesh(core_axis_name="c", subcore_axis_name="s")


@functools.partial(
    pl.kernel,
    mesh=_mesh,
    out_type=[
        jax.ShapeDtypeStruct((NP, 16), jnp.float32),
        jax.ShapeDtypeStruct((NP, 16), jnp.float32),
    ],
    scratch_types=[
        pltpu.VMEM_SHARED((NP, 16), jnp.float32),
        pltpu.VMEM((CH,), jnp.int32),
        pltpu.VMEM((CH, 16), jnp.float32),
    ],
)
def _deg_kernel(dst_hbm, ones_hbm, degA, degB, sh_deg, idx_v, ones_v):
    c = lax.axis_index("c")
    s = lax.axis_index("s")
    wid = c * 16 + s
    pltpu.sync_copy(ones_hbm.at[pl.ds(s * ROWS_T, ROWS_T)],
                    sh_deg.at[pl.ds(s * ROWS_T, ROWS_T)])
    pltpu.sync_copy(ones_hbm.at[pl.ds(0, CH)], ones_v)
    plsc.subcore_barrier()

    base0 = wid * EP_TILE

    def body(j, carry):
        pltpu.sync_copy(dst_hbm.at[pl.ds(base0 + j * CH, CH)], idx_v)
        pltpu.sync_copy(ones_v, sh_deg.at[idx_v], add=True)
        return carry

    lax.fori_loop(0, CHUNKS_PER_TILE, body, 0)
    plsc.subcore_barrier()

    @pl.when(c == 0)
    def _():
        pltpu.sync_copy(sh_deg.at[pl.ds(s * ROWS_T, ROWS_T)],
                        degA.at[pl.ds(s * ROWS_T, ROWS_T)])

    @pl.when(c == 1)
    def _():
        pltpu.sync_copy(sh_deg.at[pl.ds(s * ROWS_T, ROWS_T)],
                        degB.at[pl.ds(s * ROWS_T, ROWS_T)])


@functools.partial(
    pl.kernel,
    mesh=_mesh,
    out_type=[
        jax.ShapeDtypeStruct((NP, D), jnp.float32),
        jax.ShapeDtypeStruct((NP, D), jnp.float32),
    ],
    scratch_types=[
        pltpu.VMEM_SHARED((NP, D), jnp.float32),
        pltpu.VMEM((CH,), jnp.int32),
        pltpu.VMEM((CH,), jnp.int32),
        pltpu.VMEM((CH,), jnp.int32),
        pltpu.VMEM((CH, D), jnp.float32),
        pltpu.VMEM((CH, D), jnp.float32),
        pltpu.SemaphoreType.DMA,
        pltpu.SemaphoreType.DMA,
        pltpu.SemaphoreType.DMA,
        pltpu.SemaphoreType.DMA,
        pltpu.SemaphoreType.DMA,
    ],
)
def _scatter_step(g_hbm, src_hbm, dst_hbm, aggA, aggB, sh_agg,
                  is0, is1, idx_d, rows0, rows1,
                  semg0, semg1, sem2, semp0, semp1):
    idx_s = (is0, is1)
    rows = (rows0, rows1)
    sem_g = (semg0, semg1)
    sem_p = (semp0, semp1)
    c = lax.axis_index("c")
    s = lax.axis_index("s")
    wid = c * 16 + s
    pltpu.sync_copy(g_hbm.at[pl.ds(s * ROWS_T, ROWS_T)],
                    sh_agg.at[pl.ds(s * ROWS_T, ROWS_T)])
    plsc.subcore_barrier()
    base0 = wid * EP_TILE

    pltpu.sync_copy(src_hbm.at[pl.ds(base0, CH)], idx_s[0])
    pltpu.async_copy(g_hbm.at[idx_s[0]], rows[0], sem_g[0])
    pltpu.sync_copy(src_hbm.at[pl.ds(base0 + CH, CH)], idx_s[1])

    def chunk(j, b, last):
        bj = base0 + j * CH
        pltpu.async_copy(dst_hbm.at[pl.ds(bj, CH)], idx_d, sem2)
        pltpu.make_async_copy(g_hbm.at[idx_s[b]], rows[b], sem_g[b]).wait()
        if not last:
            pltpu.async_copy(g_hbm.at[idx_s[1 - b]], rows[1 - b], sem_g[1 - b])
            bn = base0 + jnp.minimum(j + 2, CHUNKS_PER_TILE - 1) * CH
            pltpu.async_copy(src_hbm.at[pl.ds(bn, CH)], idx_s[b], sem_p[b])
        pltpu.make_async_copy(dst_hbm.at[pl.ds(bj, CH)], idx_d, sem2).wait()
        pltpu.sync_copy(rows[b], sh_agg.at[idx_d], add=True)
        if not last:
            bn = base0 + jnp.minimum(j + 2, CHUNKS_PER_TILE - 1) * CH
            pltpu.make_async_copy(src_hbm.at[pl.ds(bn, CH)], idx_s[b],
                                  sem_p[b]).wait()

    def body(jo, carry):
        chunk(jo * 2, 0, False)
        chunk(jo * 2 + 1, 1, False)
        return carry

    lax.fori_loop(0, (CHUNKS_PER_TILE - 1) // 2, body, 0)
    chunk(CHUNKS_PER_TILE - 1, 0, True)
    plsc.subcore_barrier()

    @pl.when(c == 0)
    def _():
        pltpu.sync_copy(sh_agg.at[pl.ds(s * ROWS_T, ROWS_T)],
                        aggA.at[pl.ds(s * ROWS_T, ROWS_T)])

    @pl.when(c == 1)
    def _():
        pltpu.sync_copy(sh_agg.at[pl.ds(s * ROWS_T, ROWS_T)],
                        aggB.at[pl.ds(s * ROWS_T, ROWS_T)])


def _mlp_body(x_ref, w1_ref, b1_ref, w2_ref, b2_ref, da_ref, db_ref,
              x0_ref, g0_ref, dinv_ref):
    xb = x_ref[...]
    h = lax.dot_general(xb, w1_ref[...], (((1,), (1,)), ((), ())),
                        preferred_element_type=jnp.float32)
    h = jnp.maximum(h + b1_ref[...], 0.0)
    h = lax.dot_general(h, w2_ref[...], (((1,), (1,)), ((), ())),
                        preferred_element_type=jnp.float32) + b2_ref[...]
    deg = da_ref[...][:, :1] + db_ref[...][:, :1] - 1.0
    dinv = lax.rsqrt(deg)
    x0_ref[...] = h
    g0_ref[...] = h * dinv
    dinv_ref[...] = dinv


def _combine_body(aggA_ref, aggB_ref, g_ref, x0_ref, dinv_ref, h_ref, gn_ref):
    agg = aggA_ref[...] + aggB_ref[...] - g_ref[...]
    dinv = dinv_ref[...]
    h = (1.0 - ALPHA) * (dinv * agg) + ALPHA * x0_ref[...]
    h_ref[...] = h
    gn_ref[...] = dinv * h


_BLK = 1024
_GRID = NP // _BLK

_mlp_call = pl.pallas_call(
    _mlp_body,
    grid=(_GRID,),
    in_specs=[
        pl.BlockSpec((_BLK, D), lambda i: (i, 0)),
        pl.BlockSpec((D, D), lambda i: (0, 0)),
        pl.BlockSpec((1, D), lambda i: (0, 0)),
        pl.BlockSpec((D, D), lambda i: (0, 0)),
        pl.BlockSpec((1, D), lambda i: (0, 0)),
        pl.BlockSpec((_BLK, 16), lambda i: (i, 0)),
        pl.BlockSpec((_BLK, 16), lambda i: (i, 0)),
    ],
    out_specs=[
        pl.BlockSpec((_BLK, D), lambda i: (i, 0)),
        pl.BlockSpec((_BLK, D), lambda i: (i, 0)),
        pl.BlockSpec((_BLK, 1), lambda i: (i, 0)),
    ],
    out_shape=[
        jax.ShapeDtypeStruct((NP, D), jnp.float32),
        jax.ShapeDtypeStruct((NP, D), jnp.float32),
        jax.ShapeDtypeStruct((NP, 1), jnp.float32),
    ],
)

_combine_call = pl.pallas_call(
    _combine_body,
    grid=(_GRID,),
    in_specs=[
        pl.BlockSpec((_BLK, D), lambda i: (i, 0)),
        pl.BlockSpec((_BLK, D), lambda i: (i, 0)),
        pl.BlockSpec((_BLK, D), lambda i: (i, 0)),
        pl.BlockSpec((_BLK, D), lambda i: (i, 0)),
        pl.BlockSpec((_BLK, 1), lambda i: (i, 0)),
    ],
    out_specs=[
        pl.BlockSpec((_BLK, D), lambda i: (i, 0)),
        pl.BlockSpec((_BLK, D), lambda i: (i, 0)),
    ],
    out_shape=[
        jax.ShapeDtypeStruct((NP, D), jnp.float32),
        jax.ShapeDtypeStruct((NP, D), jnp.float32),
    ],
)


def kernel(x, edge_index, W1, b1, W2, b2):
    src = edge_index[0]
    dst = edge_index[1]
    pad = E_PAD - E
    src_p = jnp.concatenate([src, jnp.zeros((pad,), jnp.int32)])
    dst_p = jnp.concatenate([dst, jnp.full((pad,), SAC, jnp.int32)])
    ones16 = jnp.ones((NP, 16), jnp.float32)
    x_p = jnp.pad(x, ((0, NP - N), (0, 0)))

    degA, degB = _deg_kernel(dst_p, ones16)
    x0, g, dinv = _mlp_call(x_p, W1, b1.reshape(1, D), W2, b2.reshape(1, D),
                            degA, degB)
    h = x0
    for _ in range(K):
        aggA, aggB = _scatter_step(g, src_p, dst_p)
        h, g = _combine_call(aggA, aggB, g, x0, dinv)
    return h[:N]
